# baseline (speedup 1.0000x reference)
import functools

import jax
import jax.numpy as jnp
from jax import lax
from jax.experimental import pallas as pl
from jax.experimental.pallas import tpu as pltpu
from jax.experimental.pallas import tpu_sc as plsc

_N = 10000
_E = 320000
_D = 128
_G = 16
_C = 4

_NC = 2
_NS = 16
_NW = _NC * _NS
_EPT = _E // _NW
_KCH = 100
_GRP = 20
_NGRP = _EPT // (_KCH * _GRP)
_NPAD = 10240
_ZPT = _NPAD // _NS

_BLK = 2000
_NB = _N // _BLK


def _sc_mesh():
  return plsc.VectorSubcoreMesh(
      core_axis_name="c", subcore_axis_name="s",
      num_cores=_NC, num_subcores=_NS)


def _sc_count(dst_tiles):
  @functools.partial(
      pl.kernel,
      out_type=jax.ShapeDtypeStruct((_NW, _N), jnp.float32),
      mesh=_sc_mesh(),
      scratch_types=[
          pltpu.VMEM((_EPT,), jnp.int32),
          pltpu.VMEM((_N,), jnp.float32),
      ],
      compiler_params=pltpu.CompilerParams(needs_layout_passes=False),
  )
  def body(dst_hbm, out_hbm, dbuf, cnt):
    cid = lax.axis_index("c")
    sid = lax.axis_index("s")
    wid = sid * _NC + cid
    pltpu.sync_copy(dst_hbm.at[wid], dbuf)
    zeros = jnp.zeros((16,), jnp.float32)

    def zbody(i, carry):
      cnt[pl.ds(i * 16, 16)] = zeros
      return carry

    lax.fori_loop(0, _N // 16, zbody, 0)
    ones = jnp.ones((16,), jnp.float32)

    def cbody(i, carry):
      d = dbuf[pl.ds(i * 16, 16)]
      plsc.addupdate_scatter(cnt, [d], ones)
      return carry

    lax.fori_loop(0, _EPT // 16, cbody, 0)
    pltpu.sync_copy(cnt, out_hbm.at[wid])

  return body(dst_tiles)


def _sc_agg(y, src_t, dst_t):
  @functools.partial(
      pl.kernel,
      out_type=jax.ShapeDtypeStruct((_NC, _NPAD, _D), jnp.float32),
      mesh=_sc_mesh(),
      scratch_types=[
          pltpu.VMEM((_GRP, _KCH), jnp.int32),
          pltpu.VMEM((_GRP, _KCH), jnp.int32),
          pltpu.VMEM((_KCH, _D), jnp.float32),
          pltpu.VMEM((_KCH, _D), jnp.float32),
          pltpu.VMEM((_KCH, _D), jnp.float32),
          pltpu.VMEM((16, _D), jnp.float32),
          pltpu.VMEM_SHARED((_NPAD, _D), jnp.float32),
          pltpu.SemaphoreType.DMA,
          pltpu.SemaphoreType.DMA,
          pltpu.SemaphoreType.DMA,
          pltpu.SemaphoreType.DMA,
          pltpu.SemaphoreType.DMA,
          pltpu.SemaphoreType.DMA,
      ],
  )
  def body(y_hbm, src_hbm, dst_hbm, out_hbm, sidx, didx, rows0, rows1, rows2,
           zbuf, acc, sg0, sg1, sg2, ss0, ss1, ss2):
    cid = lax.axis_index("c")
    sid = lax.axis_index("s")
    wid = sid * _NC + cid
    zeros = jnp.zeros((16,), jnp.float32)

    def zb(i, carry):
      zbuf[i // 8, pl.ds((i % 8) * 16, 16)] = zeros
      return carry

    lax.fori_loop(0, 16 * (_D // 16), zb, 0)

    def zs(i, carry):
      pltpu.sync_copy(zbuf, acc.at[pl.ds(sid * _ZPT + i * 16, 16)])
      return carry

    lax.fori_loop(0, _ZPT // 16, zs, 0)
    plsc.subcore_barrier()

    rows = (rows0, rows1, rows2)
    semg = (sg0, sg1, sg2)
    sems = (ss0, ss1, ss2)

    def fire_g(r):
      pltpu.async_copy(y_hbm.at[sidx.at[r]], rows[r % 3], semg[r % 3])

    def wait_g(r):
      pltpu.make_async_copy(y_hbm.at[sidx.at[r]], rows[r % 3],
                            semg[r % 3]).wait()

    def fire_s(r):
      pltpu.async_copy(rows[r % 3], acc.at[didx.at[r]], sems[r % 3], add=True)

    def wait_s(b):
      pltpu.make_async_copy(rows[b], acc.at[didx.at[0]], sems[b]).wait()

    def gbody(g, carry):
      pltpu.sync_copy(src_hbm.at[wid, g], sidx)

      @pl.when(g > 0)
      def _():
        wait_s(0)

      fire_g(0)

      @pl.when(g > 0)
      def _():
        wait_s(1)

      fire_g(1)

      @pl.when(g > 0)
      def _():
        wait_s(2)

      pltpu.sync_copy(dst_hbm.at[wid, g], didx)
      for r in range(_GRP):
        wait_g(r)
        fire_s(r)
        if r + 2 < _GRP:
          if r >= 1:
            wait_s((r + 2) % 3)
          fire_g(r + 2)
      return carry

    lax.fori_loop(0, _NGRP, gbody, 0)
    wait_s(0)
    wait_s(1)
    wait_s(2)
    plsc.subcore_barrier()
    pltpu.sync_copy(acc.at[pl.ds(sid * _ZPT, _ZPT)],
                    out_hbm.at[cid, pl.ds(sid * _ZPT, _ZPT)])

  return body(y, src_t, dst_t)


def _tc_prep(cnts, x):
  def body(c_ref, x_ref, y_ref, dv_ref):
    deg = 1.0 + jnp.sum(c_ref[0], axis=0)
    dinv = lax.rsqrt(deg)[:, None]
    y_ref[...] = x_ref[...] * dinv
    dv_ref[...] = dinv

  return pl.pallas_call(
      body,
      grid=(_NB,),
      in_specs=[
          pl.BlockSpec((1, _NW, _BLK), lambda i: (i, 0, 0)),
          pl.BlockSpec((_BLK, _D), lambda i: (i, 0)),
      ],
      out_specs=[
          pl.BlockSpec((_BLK, _D), lambda i: (i, 0)),
          pl.BlockSpec((_BLK, 1), lambda i: (i, 0)),
      ],
      out_shape=[
          jax.ShapeDtypeStruct((_N, _D), jnp.float32),
          jax.ShapeDtypeStruct((_N, 1), jnp.float32),
      ],
  )(cnts, x)


def _tc_layer(p, y, dv, W, b):
  def body(p_ref, y_ref, dv_ref, w_ref, b_ref, o_ref):
    dinv = dv_ref[...]
    t = (p_ref[0] + p_ref[1] + y_ref[...]) * dinv
    h = jnp.dot(t, w_ref[...], preferred_element_type=jnp.float32)
    h = jnp.maximum(h + b_ref[...], 0.0)
    o_ref[...] = h * dinv

  return pl.pallas_call(
      body,
      grid=(_NB,),
      in_specs=[
          pl.BlockSpec((_NC, _BLK, _D), lambda i: (0, i, 0)),
          pl.BlockSpec((_BLK, _D), lambda i: (i, 0)),
          pl.BlockSpec((_BLK, 1), lambda i: (i, 0)),
          pl.BlockSpec((_D, _D), lambda i: (0, 0)),
          pl.BlockSpec((1, _D), lambda i: (0, 0)),
      ],
      out_specs=pl.BlockSpec((_BLK, _D), lambda i: (i, 0)),
      out_shape=jax.ShapeDtypeStruct((_N, _D), jnp.float32),
  )(p, y, dv, W, b)


def _tc_final(p, y, dv, W, b, batch_r, Wl, bl):
  def body(p_ref, y_ref, dv_ref, w_ref, b_ref, bt_ref, wl_ref, bl_ref,
           o_ref, acc, cnt):
    i = pl.program_id(0)

    @pl.when(i == 0)
    def _():
      acc[...] = jnp.zeros_like(acc)
      cnt[...] = jnp.zeros_like(cnt)

    dinv = dv_ref[...]
    t = (p_ref[0] + p_ref[1] + y_ref[...]) * dinv
    h = jnp.dot(t, w_ref[...], preferred_element_type=jnp.float32)
    h = jnp.maximum(h + b_ref[...], 0.0)
    bt = bt_ref[0]
    oh = (bt == lax.broadcasted_iota(jnp.int32, (_G, _BLK), 0))
    ohf = oh.astype(jnp.float32)
    acc[...] += jnp.dot(ohf, h, preferred_element_type=jnp.float32)
    cnt[...] += jnp.sum(ohf, axis=1)[None, :]

    @pl.when(i == _NB - 1)
    def _():
      c = jnp.maximum(cnt[0, :], 1.0)[:, None]
      pooled = acc[...] / c
      o_ref[...] = (jnp.dot(pooled, wl_ref[...],
                            preferred_element_type=jnp.float32) + bl_ref[...])

  return pl.pallas_call(
      body,
      grid=(_NB,),
      in_specs=[
          pl.BlockSpec((_NC, _BLK, _D), lambda i: (0, i, 0)),
          pl.BlockSpec((_BLK, _D), lambda i: (i, 0)),
          pl.BlockSpec((_BLK, 1), lambda i: (i, 0)),
          pl.BlockSpec((_D, _D), lambda i: (0, 0)),
          pl.BlockSpec((1, _D), lambda i: (0, 0)),
          pl.BlockSpec((1, 1, _BLK), lambda i: (i, 0, 0)),
          pl.BlockSpec((_D, _C), lambda i: (0, 0)),
          pl.BlockSpec((1, _C), lambda i: (0, 0)),
      ],
      out_specs=pl.BlockSpec((_G, _C), lambda i: (0, 0)),
      out_shape=jax.ShapeDtypeStruct((_G, _C), jnp.float32),
      scratch_shapes=[
          pltpu.VMEM((_G, _D), jnp.float32),
          pltpu.VMEM((1, _G), jnp.float32),
      ],
      compiler_params=pltpu.CompilerParams(
          dimension_semantics=("arbitrary",)),
  )(p, y, dv, W, b, batch_r, Wl, bl)


def kernel(x, edge_index, batch, W1, b1, W2, b2, Wl, bl):
  src_t = edge_index[0].reshape(_NW, _NGRP, _GRP, _KCH)
  dst_t = edge_index[1].reshape(_NW, _NGRP, _GRP, _KCH)
  dst_f = edge_index[1].reshape(_NW, _EPT)

  cnts = _sc_count(dst_f)
  cnts_r = cnts.reshape(_NW, _NB, _BLK).transpose(1, 0, 2)
  y1, dv = _tc_prep(cnts_r, x)
  p1 = _sc_agg(y1, src_t, dst_t)
  y2 = _tc_layer(p1, y1, dv, W1, b1.reshape(1, _D))
  p2 = _sc_agg(y2, src_t, dst_t)
  return _tc_final(p2, y2, dv, W2, b2.reshape(1, _D),
                   batch.reshape(_NB, 1, _BLK), Wl, bl.reshape(1, _C))


# speedup vs baseline: 1.0198x; 1.0198x over previous
import functools

import jax
import jax.numpy as jnp
from jax import lax
from jax.experimental import pallas as pl
from jax.experimental.pallas import tpu as pltpu
from jax.experimental.pallas import tpu_sc as plsc

_N = 10000
_E = 320000
_D = 128
_G = 16
_C = 4

_NC = 2
_NS = 16
_NW = _NC * _NS
_EPT = _E // _NW
_KCH = 100
_GRP = 25
_NGRP = _EPT // (_KCH * _GRP)
_NPAD = 10240
_ZPT = _NPAD // _NS

_BLK = 2000
_NB = _N // _BLK


def _sc_mesh():
  return plsc.VectorSubcoreMesh(
      core_axis_name="c", subcore_axis_name="s",
      num_cores=_NC, num_subcores=_NS)


def _sc_count(dst_tiles):
  @functools.partial(
      pl.kernel,
      out_type=jax.ShapeDtypeStruct((_NW, _N), jnp.float32),
      mesh=_sc_mesh(),
      scratch_types=[
          pltpu.VMEM((_EPT,), jnp.int32),
          pltpu.VMEM((_N,), jnp.float32),
      ],
      compiler_params=pltpu.CompilerParams(needs_layout_passes=False),
  )
  def body(dst_hbm, out_hbm, dbuf, cnt):
    cid = lax.axis_index("c")
    sid = lax.axis_index("s")
    wid = sid * _NC + cid
    pltpu.sync_copy(dst_hbm.at[wid], dbuf)
    zeros = jnp.zeros((16,), jnp.float32)

    def zbody(i, carry):
      cnt[pl.ds(i * 16, 16)] = zeros
      return carry

    lax.fori_loop(0, _N // 16, zbody, 0)
    ones = jnp.ones((16,), jnp.float32)

    def cbody(i, carry):
      d = dbuf[pl.ds(i * 16, 16)]
      plsc.addupdate_scatter(cnt, [d], ones)
      return carry

    lax.fori_loop(0, _EPT // 16, cbody, 0)
    pltpu.sync_copy(cnt, out_hbm.at[wid])

  return body(dst_tiles)


def _sc_agg(y, src_t, dst_t):
  @functools.partial(
      pl.kernel,
      out_type=jax.ShapeDtypeStruct((_NC, _NPAD, _D), jnp.float32),
      mesh=_sc_mesh(),
      scratch_types=[
          pltpu.VMEM((_GRP, _KCH), jnp.int32),
          pltpu.VMEM((_GRP, _KCH), jnp.int32),
          pltpu.VMEM((_KCH, _D), jnp.float32),
          pltpu.VMEM((_KCH, _D), jnp.float32),
          pltpu.VMEM((_KCH, _D), jnp.float32),
          pltpu.VMEM_SHARED((_NPAD, _D), jnp.float32),
          pltpu.SemaphoreType.DMA,
          pltpu.SemaphoreType.DMA,
          pltpu.SemaphoreType.DMA,
          pltpu.SemaphoreType.DMA,
          pltpu.SemaphoreType.DMA,
          pltpu.SemaphoreType.DMA,
      ],
  )
  def body(y_hbm, src_hbm, dst_hbm, out_hbm, sidx, didx, rows0, rows1, rows2,
           acc, sg0, sg1, sg2, ss0, ss1, ss2):
    cid = lax.axis_index("c")
    sid = lax.axis_index("s")
    wid = sid * _NC + cid
    zeros = jnp.zeros((16,), jnp.float32)

    def zb(i, carry):
      rows0[i // 8, pl.ds((i % 8) * 16, 16)] = zeros
      return carry

    lax.fori_loop(0, 16 * (_D // 16), zb, 0)

    def zs(i, carry):
      pltpu.sync_copy(rows0.at[pl.ds(0, 16)],
                      acc.at[pl.ds(sid * _ZPT + i * 16, 16)])
      return carry

    lax.fori_loop(0, _ZPT // 16, zs, 0)
    plsc.subcore_barrier()

    rows = (rows0, rows1, rows2)
    semg = (sg0, sg1, sg2)
    sems = (ss0, ss1, ss2)

    def fire_g(r):
      pltpu.async_copy(y_hbm.at[sidx.at[r]], rows[r % 3], semg[r % 3])

    def wait_g(r):
      pltpu.make_async_copy(y_hbm.at[sidx.at[r]], rows[r % 3],
                            semg[r % 3]).wait()

    def fire_s(r):
      pltpu.async_copy(rows[r % 3], acc.at[didx.at[r]], sems[r % 3], add=True)

    def wait_s(b):
      pltpu.make_async_copy(rows[b], acc.at[didx.at[0]], sems[b]).wait()

    def gbody(g, carry):
      pltpu.sync_copy(src_hbm.at[wid, g], sidx)

      @pl.when(g > 0)
      def _():
        wait_s(0)

      fire_g(0)

      @pl.when(g > 0)
      def _():
        wait_s(1)

      fire_g(1)

      @pl.when(g > 0)
      def _():
        wait_s(2)

      pltpu.sync_copy(dst_hbm.at[wid, g], didx)
      for r in range(_GRP):
        wait_g(r)
        fire_s(r)
        if r + 2 < _GRP:
          if r >= 1:
            wait_s((r + 2) % 3)
          fire_g(r + 2)
      return carry

    lax.fori_loop(0, _NGRP, gbody, 0)
    wait_s(0)
    wait_s(1)
    wait_s(2)
    plsc.subcore_barrier()
    pltpu.sync_copy(acc.at[pl.ds(sid * _ZPT, _ZPT)],
                    out_hbm.at[cid, pl.ds(sid * _ZPT, _ZPT)])

  return body(y, src_t, dst_t)


def _tc_prep(cnts, x):
  def body(c_ref, x_ref, y_ref, dv_ref):
    deg = 1.0 + jnp.sum(c_ref[...], axis=0)
    dinv = lax.rsqrt(deg)[:, None]
    y_ref[...] = x_ref[...] * dinv
    dv_ref[...] = dinv

  return pl.pallas_call(
      body,
      out_shape=[
          jax.ShapeDtypeStruct((_N, _D), jnp.float32),
          jax.ShapeDtypeStruct((_N, 1), jnp.float32),
      ],
  )(cnts, x)


def _tc_layer(p, y, dv, W, b):
  def body(p_ref, y_ref, dv_ref, w_ref, b_ref, o_ref):
    dinv = dv_ref[...]
    t = (p_ref[0, :_N] + p_ref[1, :_N] + y_ref[...]) * dinv
    h = jnp.dot(t, w_ref[...], preferred_element_type=jnp.float32)
    h = jnp.maximum(h + b_ref[...], 0.0)
    o_ref[...] = h * dinv

  return pl.pallas_call(
      body,
      out_shape=jax.ShapeDtypeStruct((_N, _D), jnp.float32),
  )(p, y, dv, W, b)


def _tc_final(p, y, dv, W, b, batch_r, Wl, bl):
  def body(p_ref, y_ref, dv_ref, w_ref, b_ref, bt_ref, wl_ref, bl_ref,
           o_ref):
    dinv = dv_ref[...]
    t = (p_ref[0, :_N] + p_ref[1, :_N] + y_ref[...]) * dinv
    h = jnp.dot(t, w_ref[...], preferred_element_type=jnp.float32)
    h = jnp.maximum(h + b_ref[...], 0.0)
    oh = (bt_ref[...] == lax.broadcasted_iota(jnp.int32, (_G, _N), 0))
    ohf = oh.astype(jnp.float32)
    acc = jnp.dot(ohf, h, preferred_element_type=jnp.float32)
    cnt = jnp.maximum(jnp.sum(ohf, axis=1), 1.0)[:, None]
    pooled = acc / cnt
    o_ref[...] = (jnp.dot(pooled, wl_ref[...],
                          preferred_element_type=jnp.float32) + bl_ref[...])

  return pl.pallas_call(
      body,
      out_shape=jax.ShapeDtypeStruct((_G, _C), jnp.float32),
  )(p, y, dv, W, b, batch_r, Wl, bl)


def kernel(x, edge_index, batch, W1, b1, W2, b2, Wl, bl):
  src_t = edge_index[0].reshape(_NW, _NGRP, _GRP, _KCH)
  dst_t = edge_index[1].reshape(_NW, _NGRP, _GRP, _KCH)
  dst_f = edge_index[1].reshape(_NW, _EPT)

  cnts = _sc_count(dst_f)
  y1, dv = _tc_prep(cnts, x)
  p1 = _sc_agg(y1, src_t, dst_t)
  y2 = _tc_layer(p1, y1, dv, W1, b1.reshape(1, _D))
  p2 = _sc_agg(y2, src_t, dst_t)
  return _tc_final(p2, y2, dv, W2, b2.reshape(1, _D),
                   batch.reshape(1, _N), Wl, bl.reshape(1, _C))
